# Initial kernel scaffold; baseline (speedup 1.0000x reference)
import jax
import jax.numpy as jnp
from jax.experimental import pallas as pl


def kernel(x, edge_index, edge_order, edge_type, rel_embed, id2entity_instance, W_in, b_in, W_edge, b_edge, W_sheaf, b_sheaf, W_conv, w_rel, loop_rel, bias):
    raise NotImplementedError("write your pallas kernel here")


# speedup vs baseline: 10.9165x; 10.9165x over previous
import functools
import jax
import jax.numpy as jnp
from jax.experimental import pallas as pl
from jax.experimental.pallas import tpu as pltpu


def _prep_kernel(xpad_ref, idx_ref, Win_ref, bin_ref, Wedge_ref, bedge_ref,
                 Wst_ref, Wsb_ref, bsh_ref, Wconv_ref, relpad_ref, wrel_ref,
                 xl_ref, xlsrc_ref, aent_ref, asrc_ref, ctab_ref, rout_ref,
                 inst_scr, *, R, L, pad_row):
    f32 = jnp.float32
    Win = Win_ref[...]
    Wconv = Wconv_ref[...]
    f = Wconv.shape[0]
    W2 = jnp.concatenate([Win[:, :f] @ Wconv, Win[:, f:] @ Wconv], axis=1)
    b_in = bin_ref[...]
    b2 = jnp.concatenate([b_in[:, :f] @ Wconv, b_in[:, f:] @ Wconv], axis=1)
    Wa = Win @ Wst_ref[...]
    ba = b_in @ Wst_ref[...]

    xp = xpad_ref[...]
    xl_ref[...] = xp @ W2 + b2
    aent_ref[...] = xp[:R] @ Wa + ba

    ep = relpad_ref[...] @ Wedge_ref[...] + bedge_ref[...]
    ctab_ref[...] = ep @ Wsb_ref[...] + bsh_ref[...]
    rout_ref[...] = relpad_ref[...] @ wrel_ref[...]

    def body(i, _):
        acc = jnp.zeros((1, xp.shape[1]), f32)
        cnt = jnp.zeros((), f32)
        for l in range(L):
            j = idx_ref[i, l]
            acc = acc + xpad_ref[pl.ds(j, 1), :]
            cnt = cnt + jnp.where(j != pad_row, 1.0, 0.0).astype(f32)
        inst_scr[pl.ds(i, 1), :] = acc / cnt
        return 0

    jax.lax.fori_loop(0, R, body, 0)
    inst = inst_scr[...]
    xlsrc_ref[...] = inst @ W2 + b2
    asrc_ref[...] = inst @ Wa + ba


def _deg_kernel(srow_ref, trow_ref, degs_ref, degt_ref, *, RP):
    i = pl.program_id(0)
    C = srow_ref.shape[2]
    bf = jnp.bfloat16

    @pl.when(i == 0)
    def _():
        degs_ref[...] = jnp.zeros_like(degs_ref)
        degt_ref[...] = jnp.zeros_like(degt_ref)

    iota = jax.lax.broadcasted_iota(jnp.int32, (RP, C), 0)
    ones = jnp.ones((C, 1), bf)
    s_ohT = (iota == srow_ref[0]).astype(bf)
    t_ohT = (iota == trow_ref[0]).astype(bf)
    dn = (((0,), (0,)), ((), ()))
    degs_ref[...] += jax.lax.dot_general(
        s_ohT, ones, (((1,), (0,)), ((), ())), preferred_element_type=jnp.float32)
    degt_ref[...] += jax.lax.dot_general(
        t_ohT, ones, (((1,), (0,)), ((), ())), preferred_element_type=jnp.float32)
    del dn


def _edges_kernel(scol_ref, trow_ref, ecol_ref, asrc_ref, atgt_ref, ctab_ref,
                  xlsrc_ref, degs_ref, acc_ref, z_scr, *, RP, NT):
    i = pl.program_id(0)
    C = scol_ref.shape[1]
    bf = jnp.bfloat16
    f32 = jnp.float32
    F2 = xlsrc_ref.shape[1]

    @pl.when(i == 0)
    def _():
        acc_ref[...] = jnp.zeros_like(acc_ref)
        ds = degs_ref[...]
        dinv_s = jnp.where(ds > 0, jax.lax.rsqrt(ds), 0.0)
        z_scr[...] = (xlsrc_ref[...] * dinv_s).astype(bf)

    s_col = scol_ref[0]
    e_col = ecol_ref[0]
    t_row = trow_ref[0]

    iota_r = jax.lax.broadcasted_iota(jnp.int32, (C, RP), 1)
    iota_e = jax.lax.broadcasted_iota(jnp.int32, (C, NT), 1)
    iota_t = jax.lax.broadcasted_iota(jnp.int32, (RP, C), 0)
    s_oh = (iota_r == s_col).astype(bf)
    e_oh = (iota_e == e_col).astype(bf)
    t_ohT = (iota_t == t_row).astype(bf)

    ec = jnp.dot(e_oh, ctab_ref[...], preferred_element_type=f32)
    p = jnp.tanh(jnp.dot(s_oh, asrc_ref[...], preferred_element_type=f32) + ec)
    qpre = jax.lax.dot_general(t_ohT, atgt_ref[...], (((0,), (0,)), ((), ())),
                               preferred_element_type=f32)
    q = jnp.tanh(qpre + ec)
    zg = jnp.dot(s_oh, z_scr[...], preferred_element_type=f32)
    rhs = jnp.concatenate([(p * q) * zg, q * q], axis=1).astype(bf)
    acc_ref[...] += jnp.dot(t_ohT, rhs, preferred_element_type=f32)


def _final_kernel(xl_ref, acc_ref, degt_ref, bias_ref, out_ref, *, R, N):
    f32 = jnp.float32
    F2 = xl_ref.shape[1]
    scale = 1.0 / (1.0 + 1e-5) ** 0.5
    dt = degt_ref[pl.ds(0, R), :]
    dinv_t = jnp.where(dt > 0, jax.lax.rsqrt(dt), 0.0)
    xl_t = xl_ref[pl.ds(0, R), :]
    cross = acc_ref[pl.ds(0, R), :F2]
    sq = acc_ref[pl.ds(0, R), F2:]
    y = 0.5 * dinv_t * (cross + sq * (dinv_t * xl_t))
    v = xl_t - y
    bias = bias_ref[...]
    out_ref[pl.ds(0, R), :] = (jnp.where(v > 0, v, jnp.exp(jnp.minimum(v, 0.0)) - 1.0) + bias) * scale
    v2 = xl_ref[pl.ds(R, N - R), :]
    out_ref[pl.ds(R, N - R), :] = (jnp.where(v2 > 0, v2, jnp.exp(jnp.minimum(v2, 0.0)) - 1.0) + bias) * scale


def _run(x, edge_index, edge_type, rel_embed, id2entity_instance,
         W_in, b_in, W_edge, b_edge, W_sheaf, b_sheaf, W_conv, w_rel,
         loop_rel, bias, interpret=False):
    f32 = jnp.float32
    N, F = x.shape
    R, L = id2entity_instance.shape
    E = edge_index.shape[1]
    f = W_conv.shape[0]
    F2 = 2 * f
    num_rels = rel_embed.shape[0]

    NPAD = ((N + L + 7) // 8) * 8
    pad_row = N
    xpad = jnp.zeros((NPAD, F), f32).at[:N].set(x.astype(f32))
    idx = jnp.where(id2entity_instance < 0, pad_row,
                    id2entity_instance).astype(jnp.int32)
    RP = ((R + 127) // 128) * 128
    NT = ((num_rels + 1 + 127) // 128) * 128
    relpad = jnp.zeros((NT, F), f32)
    relpad = relpad.at[:num_rels].set(rel_embed.astype(f32))
    relpad = relpad.at[num_rels].set(loop_rel.astype(f32).reshape(F))
    Wst = jnp.repeat(W_sheaf[:F].astype(f32), f, axis=1)
    Wsb = jnp.repeat(W_sheaf[F:].astype(f32), f, axis=1)
    bsh = jnp.repeat(b_sheaf.astype(f32), f, axis=0).reshape(1, F2)
    C = 640
    while E % C != 0:
        C //= 2
    steps = E // C
    t_all = edge_index[0].astype(jnp.int32)
    s_all = edge_index[1].astype(jnp.int32)
    e_all = edge_type.astype(jnp.int32)
    s_col = s_all.reshape(steps, C, 1)
    e_col = e_all.reshape(steps, C, 1)
    s_row = s_all.reshape(steps, 1, C)
    t_row = t_all.reshape(steps, 1, C)

    kp = functools.partial(_prep_kernel, R=R, L=L, pad_row=pad_row)
    xl, xlsrc, aent, asrc, ctab, rout = pl.pallas_call(
        kp,
        in_specs=[
            pl.BlockSpec(memory_space=pltpu.VMEM),
            pl.BlockSpec(memory_space=pltpu.SMEM),
        ] + [pl.BlockSpec(memory_space=pltpu.VMEM)] * 10,
        out_shape=[
            jax.ShapeDtypeStruct((NPAD, F2), f32),
            jax.ShapeDtypeStruct((R, F2), f32),
            jax.ShapeDtypeStruct((R, F2), f32),
            jax.ShapeDtypeStruct((R, F2), f32),
            jax.ShapeDtypeStruct((NT, F2), f32),
            jax.ShapeDtypeStruct((NT, F), f32),
        ],
        scratch_shapes=[pltpu.VMEM((R, F), f32)],
        interpret=interpret,
    )(xpad, idx, W_in.astype(f32), b_in.astype(f32).reshape(1, F2),
      W_edge.astype(f32), b_edge.astype(f32).reshape(1, F2),
      Wst, Wsb, bsh, W_conv.astype(f32), relpad, w_rel.astype(f32))

    kd = functools.partial(_deg_kernel, RP=RP)
    deg_s, deg_t = pl.pallas_call(
        kd,
        grid=(steps,),
        in_specs=[
            pl.BlockSpec((1, 1, C), lambda i: (i, 0, 0)),
            pl.BlockSpec((1, 1, C), lambda i: (i, 0, 0)),
        ],
        out_specs=[
            pl.BlockSpec((RP, 1), lambda i: (0, 0)),
            pl.BlockSpec((RP, 1), lambda i: (0, 0)),
        ],
        out_shape=[
            jax.ShapeDtypeStruct((RP, 1), f32),
            jax.ShapeDtypeStruct((RP, 1), f32),
        ],
        interpret=interpret,
    )(s_row, t_row)

    bf = jnp.bfloat16
    asrc_p = jnp.zeros((RP, F2), bf).at[:R].set(asrc.astype(bf))
    atgt_p = jnp.zeros((RP, F2), bf).at[:R].set(aent.astype(bf))
    xlsrc_p = jnp.zeros((RP, F2), f32).at[:R].set(xlsrc)
    ke = functools.partial(_edges_kernel, RP=RP, NT=NT)
    acc = pl.pallas_call(
        ke,
        grid=(steps,),
        in_specs=[
            pl.BlockSpec((1, C, 1), lambda i: (i, 0, 0)),
            pl.BlockSpec((1, 1, C), lambda i: (i, 0, 0)),
            pl.BlockSpec((1, C, 1), lambda i: (i, 0, 0)),
            pl.BlockSpec((RP, F2), lambda i: (0, 0)),
            pl.BlockSpec((RP, F2), lambda i: (0, 0)),
            pl.BlockSpec((NT, F2), lambda i: (0, 0)),
            pl.BlockSpec((RP, F2), lambda i: (0, 0)),
            pl.BlockSpec((RP, 1), lambda i: (0, 0)),
        ],
        out_specs=pl.BlockSpec((RP, 2 * F2), lambda i: (0, 0)),
        out_shape=jax.ShapeDtypeStruct((RP, 2 * F2), f32),
        scratch_shapes=[pltpu.VMEM((RP, F2), bf)],
        interpret=interpret,
    )(s_col, t_row, e_col, asrc_p, atgt_p, ctab.astype(bf), xlsrc_p, deg_s)

    kf = functools.partial(_final_kernel, R=R, N=N)
    out = pl.pallas_call(
        kf,
        in_specs=[pl.BlockSpec(memory_space=pltpu.VMEM)] * 4,
        out_shape=jax.ShapeDtypeStruct((N, F2), f32),
        interpret=interpret,
    )(xl, acc, deg_t, bias.astype(f32).reshape(1, F2))

    return out, rout[:num_rels]


@jax.jit
def kernel(x, edge_index, edge_order, edge_type, rel_embed,
           id2entity_instance, W_in, b_in, W_edge, b_edge, W_sheaf, b_sheaf,
           W_conv, w_rel, loop_rel, bias):
    del edge_order
    return _run(x, edge_index, edge_type, rel_embed, id2entity_instance,
                W_in, b_in, W_edge, b_edge, W_sheaf, b_sheaf, W_conv, w_rel,
                loop_rel, bias)
